# baseline (speedup 1.0000x reference)
import jax
import jax.numpy as jnp
from jax import lax
from jax.experimental import pallas as pl
from jax.experimental.pallas import tpu as pltpu

B, K, L, H = 64, 64, 64, 64

_DNT = (((1,), (1,)), ((), ()))


def _mmT(x, w):
    return lax.dot_general(x, w, _DNT, preferred_element_type=jnp.float32)


def _mmT16(x, w):
    return lax.dot_general(x.astype(jnp.bfloat16), w.astype(jnp.bfloat16),
                           _DNT, preferred_element_type=jnp.float32)


def _fused_kernel(h_ue_ref, h_ap_ref, e_ref,
                  wa1_ref, ba1_ref, wa2_ref, ba2_ref,
                  wu1_ref, bu1_ref, wu2_ref, bu2_ref,
                  wih_ue_ref, bih_ue_ref, whh_ue_ref, bhh_ue_ref,
                  wih_ap_ref, bih_ap_ref, whh_ap_ref, bhh_ap_ref,
                  we1_ref, be1_ref, we2_ref, be2_ref,
                  h_ue_out_ref, h_ap_out_ref, e_out_ref):
    hu = h_ue_ref[0]
    ha = h_ap_ref[0]
    e2 = e_ref[0]

    a_src = _mmT(ha, wa1_ref[:, :H])
    t = _mmT16(e2, wa1_ref[:, H:])
    t = t.reshape(L, K, H) + a_src[:, None, :] + ba1_ref[...]
    t = jax.nn.relu(t).reshape(L * K, H)
    m = _mmT16(t, wa2_ref[...])
    m_ue = m.reshape(L, K, H).sum(axis=0) * (1.0 / L) + ba2_ref[...]

    u_src = _mmT(hu, wu1_ref[:, :H])
    t = _mmT16(e2, wu1_ref[:, H:])
    t = t.reshape(L, K, H) + u_src[None, :, :] + bu1_ref[...]
    t = jax.nn.relu(t).reshape(L * K, H)
    m = _mmT16(t, wu2_ref[...])
    m_ap = m.reshape(L, K, H).sum(axis=1) * (1.0 / K) + bu2_ref[...]

    def gru(x, h, wih_ref, bih_ref, whh_ref, bhh_ref):
        gi = _mmT(x, wih_ref[...]) + bih_ref[...]
        gh = _mmT(h, whh_ref[...]) + bhh_ref[...]
        r = jax.nn.sigmoid(gi[:, :H] + gh[:, :H])
        z = jax.nn.sigmoid(gi[:, H:2 * H] + gh[:, H:2 * H])
        n = jnp.tanh(gi[:, 2 * H:] + r * gh[:, 2 * H:])
        return (1.0 - z) * n + z * h

    hu_new = gru(m_ue, hu, wih_ue_ref, bih_ue_ref, whh_ue_ref, bhh_ue_ref)
    ha_new = gru(m_ap, ha, wih_ap_ref, bih_ap_ref, whh_ap_ref, bhh_ap_ref)
    h_ue_out_ref[0] = hu_new
    h_ap_out_ref[0] = ha_new

    s_u = _mmT(hu_new, we1_ref[:, :H])
    s_a = _mmT(ha_new, we1_ref[:, H:2 * H])
    t = _mmT16(e2, we1_ref[:, 2 * H:])
    t = t.reshape(L, K, H) + s_u[None, :, :] + s_a[:, None, :] + be1_ref[...]
    t = jax.nn.relu(t).reshape(L * K, H)
    e_out_ref[0] = _mmT16(t, we2_ref[...]) + be2_ref[...]


def kernel(h_ue, h_ap, e, W_a2u_1, b_a2u_1, W_a2u_2, b_a2u_2,
           W_u2a_1, b_u2a_1, W_u2a_2, b_u2a_2,
           Wih_ue, bih_ue, Whh_ue, bhh_ue, Wih_ap, bih_ap, Whh_ap, bhh_ap,
           W_e_1, b_e_1, W_e_2, b_e_2):
    batch3 = lambda s: pl.BlockSpec((1,) + s, lambda b: (b, 0, 0))
    fixed = lambda s: pl.BlockSpec(s, lambda b: (0,) * len(s))

    out_shapes = (
        jax.ShapeDtypeStruct((B, K, H), jnp.float32),
        jax.ShapeDtypeStruct((B, L, H), jnp.float32),
        jax.ShapeDtypeStruct((B, L * K, H), jnp.float32),
    )

    in_specs = [
        batch3((K, H)), batch3((L, H)), batch3((L * K, H)),
        fixed((H, 2 * H)), fixed((1, H)), fixed((H, H)), fixed((1, H)),
        fixed((H, 2 * H)), fixed((1, H)), fixed((H, H)), fixed((1, H)),
        fixed((3 * H, H)), fixed((1, 3 * H)), fixed((3 * H, H)), fixed((1, 3 * H)),
        fixed((3 * H, H)), fixed((1, 3 * H)), fixed((3 * H, H)), fixed((1, 3 * H)),
        fixed((H, 3 * H)), fixed((1, H)), fixed((H, H)), fixed((1, H)),
    ]

    return pl.pallas_call(
        _fused_kernel,
        grid=(B,),
        in_specs=in_specs,
        out_specs=[batch3((K, H)), batch3((L, H)), batch3((L * K, H))],
        out_shape=out_shapes,
        compiler_params=pltpu.CompilerParams(
            dimension_semantics=("arbitrary",),
        ),
    )(h_ue, h_ap, e,
      W_a2u_1, b_a2u_1.reshape(1, H), W_a2u_2, b_a2u_2.reshape(1, H),
      W_u2a_1, b_u2a_1.reshape(1, H), W_u2a_2, b_u2a_2.reshape(1, H),
      Wih_ue, bih_ue.reshape(1, 3 * H), Whh_ue, bhh_ue.reshape(1, 3 * H),
      Wih_ap, bih_ap.reshape(1, 3 * H), Whh_ap, bhh_ap.reshape(1, 3 * H),
      W_e_1, b_e_1.reshape(1, H), W_e_2, b_e_2.reshape(1, H))
